# baseline (speedup 1.0000x reference)
import functools

import jax
import jax.numpy as jnp
from jax import lax
from jax.experimental import pallas as pl
from jax.experimental.pallas import tpu as pltpu
from jax.experimental.pallas import tpu_sc as plsc

_NC = 2
_NS = 16
_NW = _NC * _NS
_CH = 16
_NBUF = 4


def kernel(x, pos_table):
    batch, seq, dim = x.shape
    rows_per_w = seq // _NW
    n_ch = rows_per_w // _CH
    n_it = n_ch * batch
    lanes = dim // 16

    x2 = x.reshape(batch * seq, dim)
    mesh = plsc.VectorSubcoreMesh(core_axis_name="c", subcore_axis_name="s")

    @functools.partial(
        pl.kernel,
        out_type=jax.ShapeDtypeStruct((batch * seq, dim), jnp.float32),
        mesh=mesh,
        scratch_types=[
            pltpu.VMEM((_CH, dim), jnp.float32),
            pltpu.VMEM((_NBUF, _CH, dim), jnp.float32),
            pltpu.VMEM((_NBUF, _CH, dim), jnp.float32),
            pltpu.SemaphoreType.DMA,
            [pltpu.SemaphoreType.DMA] * _NBUF,
            [pltpu.SemaphoreType.DMA] * _NBUF,
        ],
    )
    def sc_add(x_hbm, pos_hbm, out_hbm, pbuf, xbuf, obuf, sp, sx, so):
        wid = lax.axis_index("s") * _NC + lax.axis_index("c")
        base = wid * rows_per_w

        def x_row(it):
            j = lax.shift_right_logical(it, 2)
            b = lax.bitwise_and(it, 3)
            return b * seq + base + j * _CH

        def fire_in(it, slot):
            pltpu.async_copy(x_hbm.at[pl.ds(x_row(it), _CH), :],
                             xbuf.at[slot], sx[slot])

        def fire_pos(j):
            pltpu.async_copy(pos_hbm.at[pl.ds(base + j * _CH, _CH), :],
                             pbuf, sp)

        def wait_pos(j):
            pltpu.make_async_copy(pos_hbm.at[pl.ds(base + j * _CH, _CH), :],
                                  pbuf, sp).wait()

        for p in range(_NBUF):
            fire_in(p, p)
        fire_pos(0)

        @pl.loop(0, n_it // _NBUF)
        def _pipe(g):
            for ph in range(_NBUF):
                it = g * _NBUF + ph
                j = lax.shift_right_logical(it, 2)
                b = lax.bitwise_and(it, 3)
                xrow = x_row(it)

                pltpu.make_async_copy(x_hbm.at[pl.ds(xrow, _CH), :],
                                      xbuf.at[ph], sx[ph]).wait()

                @pl.when(b == 0)
                def _():
                    wait_pos(j)

                @pl.when(it >= _NBUF)
                def _():
                    orow = x_row(it - _NBUF)
                    pltpu.make_async_copy(obuf.at[ph],
                                          out_hbm.at[pl.ds(orow, _CH), :],
                                          so[ph]).wait()

                @pl.loop(0, _CH)
                def _rows(r):
                    for i in range(lanes):
                        sl = pl.ds(i * 16, 16)
                        obuf[ph, r, sl] = xbuf[ph, r, sl] + pbuf[r, sl]

                @pl.when(jnp.logical_and(b == 3, j + 1 < n_ch))
                def _():
                    fire_pos(j + 1)

                pltpu.async_copy(obuf.at[ph],
                                 out_hbm.at[pl.ds(xrow, _CH), :], so[ph])

                @pl.when(it + _NBUF < n_it)
                def _():
                    fire_in(it + _NBUF, ph)

        for ph in range(_NBUF):
            it = n_it - _NBUF + ph
            xrow = x_row(it)
            pltpu.make_async_copy(obuf.at[ph],
                                  out_hbm.at[pl.ds(xrow, _CH), :],
                                  so[ph]).wait()

    out = sc_add(x2, pos_table)
    return out.reshape(batch, seq, dim)


# speedup vs baseline: 1.7927x; 1.7927x over previous
import functools

import jax
import jax.numpy as jnp
from jax import lax
from jax.experimental import pallas as pl
from jax.experimental.pallas import tpu as pltpu
from jax.experimental.pallas import tpu_sc as plsc

_NC = 2
_NS = 16
_NW = _NC * _NS
_CH = 32


def kernel(x, pos_table):
    batch, seq, dim = x.shape
    rows_per_w = seq // _NW
    n_ch = rows_per_w // _CH
    n_it = n_ch * batch
    lanes = dim // 16

    x2 = x.reshape(batch * seq, dim)
    mesh = plsc.VectorSubcoreMesh(core_axis_name="c", subcore_axis_name="s")

    @functools.partial(
        pl.kernel,
        out_type=jax.ShapeDtypeStruct((batch * seq, dim), jnp.float32),
        mesh=mesh,
        scratch_types=[
            pltpu.VMEM((_CH, dim), jnp.float32),
            pltpu.VMEM((2, _CH, dim), jnp.float32),
            pltpu.VMEM((2, _CH, dim), jnp.float32),
            pltpu.SemaphoreType.DMA,
            pltpu.SemaphoreType.DMA,
            pltpu.SemaphoreType.DMA,
            pltpu.SemaphoreType.DMA,
            pltpu.SemaphoreType.DMA,
        ],
    )
    def sc_add(x_hbm, pos_hbm, out_hbm, pbuf, xbuf, obuf,
               sx0, sx1, sp, so0, so1):
        wid = lax.axis_index("s") * _NC + lax.axis_index("c")
        base = wid * rows_per_w
        sx = (sx0, sx1)
        so = (so0, so1)

        def x_row(it):
            j = lax.shift_right_logical(it, 2)
            b = lax.bitwise_and(it, 3)
            return b * seq + base + j * _CH

        def fire_in(it, slot):
            pltpu.async_copy(x_hbm.at[pl.ds(x_row(it), _CH), :],
                             xbuf.at[slot], sx[slot])

        def fire_pos(j):
            pltpu.async_copy(pos_hbm.at[pl.ds(base + j * _CH, _CH), :],
                             pbuf, sp)

        def wait_pos(j):
            pltpu.make_async_copy(pos_hbm.at[pl.ds(base + j * _CH, _CH), :],
                                  pbuf, sp).wait()

        fire_in(0, 0)
        fire_in(1, 1)
        fire_pos(0)

        @pl.loop(0, n_it // 2)
        def _pipe(g):
            for ph in range(2):
                it = g * 2 + ph
                j = lax.shift_right_logical(it, 2)
                b = lax.bitwise_and(it, 3)
                xrow = x_row(it)

                pltpu.make_async_copy(x_hbm.at[pl.ds(xrow, _CH), :],
                                      xbuf.at[ph], sx[ph]).wait()

                @pl.when(b == 0)
                def _():
                    wait_pos(j)

                @pl.when(it >= 2)
                def _():
                    orow = x_row(it - 2)
                    pltpu.make_async_copy(obuf.at[ph],
                                          out_hbm.at[pl.ds(orow, _CH), :],
                                          so[ph]).wait()

                half = _CH // 2

                @pl.loop(0, half)
                def _rows_a(r):
                    for i in range(lanes):
                        sl = pl.ds(i * 16, 16)
                        obuf[ph, r, sl] = xbuf[ph, r, sl] + pbuf[r, sl]

                pltpu.async_copy(obuf.at[ph, pl.ds(0, half), :],
                                 out_hbm.at[pl.ds(xrow, half), :], so[ph])

                @pl.loop(half, _CH)
                def _rows_b(r):
                    for i in range(lanes):
                        sl = pl.ds(i * 16, 16)
                        obuf[ph, r, sl] = xbuf[ph, r, sl] + pbuf[r, sl]

                pltpu.async_copy(obuf.at[ph, pl.ds(half, half), :],
                                 out_hbm.at[pl.ds(xrow + half, half), :],
                                 so[ph])

                @pl.when(jnp.logical_and(b == 3, j + 1 < n_ch))
                def _():
                    fire_pos(j + 1)

                @pl.when(it + 2 < n_it)
                def _():
                    fire_in(it + 2, ph)

        for ph in range(2):
            it = n_it - 2 + ph
            xrow = x_row(it)
            pltpu.make_async_copy(obuf.at[ph],
                                  out_hbm.at[pl.ds(xrow, _CH), :],
                                  so[ph]).wait()

    out = sc_add(x2, pos_table)
    return out.reshape(batch, seq, dim)


# speedup vs baseline: 2.2375x; 1.2481x over previous
import functools

import jax
import jax.numpy as jnp
from jax import lax
from jax.experimental import pallas as pl
from jax.experimental.pallas import tpu as pltpu
from jax.experimental.pallas import tpu_sc as plsc

_NC = 2
_NS = 16
_NW = _NC * _NS
_CH = 32


def kernel(x, pos_table):
    batch, seq, dim = x.shape
    rows_per_w = seq // _NW
    n_ch = rows_per_w // _CH
    n_it = n_ch * batch
    lanes = dim // 16

    x2 = x.reshape(batch * seq, dim)
    mesh = plsc.VectorSubcoreMesh(core_axis_name="c", subcore_axis_name="s")

    @functools.partial(
        pl.kernel,
        out_type=jax.ShapeDtypeStruct((batch * seq, dim), jnp.float32),
        mesh=mesh,
        scratch_types=[
            pltpu.VMEM((_CH, dim), jnp.float32),
            pltpu.VMEM((2, _CH, dim), jnp.float32),
            pltpu.VMEM((2, _CH, dim), jnp.float32),
            pltpu.SemaphoreType.DMA,
            pltpu.SemaphoreType.DMA,
            pltpu.SemaphoreType.DMA,
            pltpu.SemaphoreType.DMA,
            pltpu.SemaphoreType.DMA,
        ],
    )
    def sc_add(x_hbm, pos_hbm, out_hbm, pbuf, xbuf, obuf,
               sx0, sx1, sp, so0, so1):
        wid = lax.axis_index("s") * _NC + lax.axis_index("c")
        base = wid * rows_per_w
        sx = (sx0, sx1)
        so = (so0, so1)

        def x_row(it):
            j = lax.shift_right_logical(it, 2)
            b = lax.bitwise_and(it, 3)
            return b * seq + base + j * _CH

        def fire_in(it, slot):
            pltpu.async_copy(x_hbm.at[pl.ds(x_row(it), _CH), :],
                             xbuf.at[slot], sx[slot])

        def fire_pos(j):
            pltpu.async_copy(pos_hbm.at[pl.ds(base + j * _CH, _CH), :],
                             pbuf, sp)

        def wait_pos(j):
            pltpu.make_async_copy(pos_hbm.at[pl.ds(base + j * _CH, _CH), :],
                                  pbuf, sp).wait()

        fire_in(0, 0)
        fire_in(1, 1)
        fire_pos(0)

        @pl.loop(0, n_it // 2)
        def _pipe(g):
            for ph in range(2):
                it = g * 2 + ph
                j = lax.shift_right_logical(it, 2)
                b = lax.bitwise_and(it, 3)
                xrow = x_row(it)

                pltpu.make_async_copy(x_hbm.at[pl.ds(xrow, _CH), :],
                                      xbuf.at[ph], sx[ph]).wait()

                @pl.when(b == 0)
                def _():
                    wait_pos(j)

                @pl.when(it >= 2)
                def _():
                    orow = x_row(it - 2)
                    pltpu.make_async_copy(obuf.at[ph],
                                          out_hbm.at[pl.ds(orow, _CH), :],
                                          so[ph]).wait()

                @pl.loop(0, _CH)
                def _rows(r):
                    for i in range(lanes):
                        sl = pl.ds(i * 16, 16)
                        obuf[ph, r, sl] = xbuf[ph, r, sl] + pbuf[r, sl]

                @pl.when(jnp.logical_and(b == 3, j + 1 < n_ch))
                def _():
                    fire_pos(j + 1)

                pltpu.async_copy(obuf.at[ph],
                                 out_hbm.at[pl.ds(xrow, _CH), :], so[ph])

                @pl.when(it + 2 < n_it)
                def _():
                    fire_in(it + 2, ph)

        for ph in range(2):
            it = n_it - 2 + ph
            xrow = x_row(it)
            pltpu.make_async_copy(obuf.at[ph],
                                  out_hbm.at[pl.ds(xrow, _CH), :],
                                  so[ph]).wait()

    out = sc_add(x2, pos_table)
    return out.reshape(batch, seq, dim)
